# baseline (speedup 1.0000x reference)
import functools

import jax
import jax.numpy as jnp
from jax import lax
from jax.experimental import pallas as pl
from jax.experimental.pallas import tpu as pltpu
from jax.experimental.pallas import tpu_sc as plsc

D_EMB = 64
NUM_WORKERS = 32
N_ROWS = 819200
B_PER_W = N_ROWS // NUM_WORKERS
CHUNK = 512
N_CHUNKS = B_PER_W // CHUNK
GGRP = 128
NGATH = CHUNK // GGRP
LANES = 16

_SPLAT_DNUMS = lax.GatherDimensionNumbers(
    offset_dims=(), collapsed_slice_dims=(0,), start_index_map=(0,))


def _splat_lane(vec, lane):
    idx = jnp.full((LANES, 1), lane, jnp.int32)
    return lax.gather(vec, idx, _SPLAT_DNUMS, slice_sizes=(1,),
                      mode=lax.GatherScatterMode.PROMISE_IN_BOUNDS)


@functools.partial(
    pl.kernel,
    mesh=plsc.VectorSubcoreMesh(core_axis_name="c", subcore_axis_name="s"),
    compiler_params=pltpu.CompilerParams(use_tc_tiling_on_sc=False),
    out_type=jax.ShapeDtypeStruct((N_ROWS, D_EMB), jnp.float32),
    scratch_types=[
        pltpu.VMEM((CHUNK,), jnp.int32),
        pltpu.VMEM((CHUNK,), jnp.int32),
        pltpu.VMEM((CHUNK,), jnp.int32),
        pltpu.VMEM((CHUNK,), jnp.int32),
        pltpu.VMEM((CHUNK, D_EMB), jnp.float32),
        pltpu.VMEM((CHUNK, D_EMB), jnp.float32),
        pltpu.SemaphoreType.DMA,
        pltpu.SemaphoreType.DMA,
        pltpu.SemaphoreType.DMA,
        pltpu.SemaphoreType.DMA,
    ],
)
def _embed(x_ref, mask_ref, table_ref, out_ref,
           idx0, idx1, msk0, msk1, rows0, rows1,
           sem_i0, sem_i1, sem_g, sem_w):
    wid = lax.axis_index("s") * 2 + lax.axis_index("c")
    base_w = wid * B_PER_W
    idx = (idx0, idx1)
    msk = (msk0, msk1)
    rows = (rows0, rows1)
    sem_i = (sem_i0, sem_i1)

    def stage_copies(g, b):
        base = base_w + g * CHUNK
        return (
            pltpu.make_async_copy(x_ref.at[pl.ds(base, CHUNK)], idx[b], sem_i[b]),
            pltpu.make_async_copy(mask_ref.at[pl.ds(base, CHUNK)], msk[b], sem_i[b]),
        )

    GATHER_ON = True

    def gather_copies(b):
        if not GATHER_ON:
            return []
        return [
            pltpu.make_async_copy(
                table_ref.at[idx[b].at[pl.ds(j * GGRP, GGRP)]],
                rows[b].at[pl.ds(j * GGRP, GGRP)],
                sem_g,
            )
            for j in range(NGATH)
        ]

    def wb_copy(g, b):
        base = base_w + g * CHUNK
        return pltpu.make_async_copy(rows[b], out_ref.at[pl.ds(base, CHUNK)], sem_w)

    def multiply(b):
        def grp_body(q, c2):
            mvec = msk[b][pl.ds(q * LANES, LANES)].astype(jnp.float32)
            for r16 in range(LANES):
                m = _splat_lane(mvec, r16)
                r = q * LANES + r16
                for s in range(D_EMB // LANES):
                    sl = rows[b][r, pl.ds(s * LANES, LANES)]
                    rows[b][r, pl.ds(s * LANES, LANES)] = sl * m
            return c2
        lax.fori_loop(0, CHUNK // LANES, grp_body, 0)

    for c in stage_copies(0, 0):
        c.start()
    for c in stage_copies(1, 1):
        c.start()
    for c in stage_copies(0, 0):
        c.wait()
    for c in gather_copies(0):
        c.start()

    def body(gi, carry):
        for b in (0, 1):
            g = 2 * gi + b
            for c in gather_copies(b):
                c.wait()
            if b == 0:
                @pl.when(gi >= 1)
                def _():
                    wb_copy(g - 1, 1).wait()
                for c in stage_copies(g + 1, 1):
                    c.wait()
                for c in gather_copies(1):
                    c.start()
            else:
                @pl.when(gi <= (N_CHUNKS - 2 - b) // 2)
                def _():
                    wb_copy(g - 1, 0).wait()
                    for c in stage_copies(g + 1, 0):
                        c.wait()
                    for c in gather_copies(0):
                        c.start()
            multiply(b)
            @pl.when(gi <= (N_CHUNKS - 3 - b) // 2)
            def _():
                for c in stage_copies(g + 2, b):
                    c.start()
            wb_copy(g, b).start()
        return carry

    lax.fori_loop(0, N_CHUNKS // 2, body, 0)
    wb_copy(N_CHUNKS - 2, 0).wait()
    wb_copy(N_CHUNKS - 1, 1).wait()


def kernel(x, mask, table, predict):
    b, h = x.shape
    n = b * h
    xf = x.reshape(n).astype(jnp.int32)
    mf = mask.reshape(n).astype(jnp.int32)
    out = _embed(xf, mf, table)
    return out.reshape(b, h, D_EMB)
